# baseline (speedup 1.0000x reference)
import jax
import jax.numpy as jnp
from jax import lax
from jax.experimental import pallas as pl
from jax.experimental.pallas import tpu as pltpu
from jax.experimental.pallas import tpu_sc as plsc

NUM_FACTOR = 32
BATCH = 16384
NC = 2
NS = 16
LANES = 16
NW = NC * NS
B_PER_W = BATCH // NW
G = 4
NGROUP = B_PER_W // G
TCOL = 128


def _gmf_body(users_hbm, items_hbm, tab_u_hbm, tab_i_hbm, wb_hbm,
              out_hbm, idx_u, idx_i, ubuf, ibuf, out_v, wb_v,
              sem_u, sem_i):
    wid = lax.axis_index("s") * NC + lax.axis_index("c")

    pltpu.sync_copy(users_hbm.at[wid], idx_u.at[pl.ds(0, B_PER_W)])
    pltpu.sync_copy(items_hbm.at[wid], idx_i.at[pl.ds(0, B_PER_W)])
    pltpu.sync_copy(wb_hbm, wb_v)
    idx_u[pl.ds(B_PER_W, LANES)] = jnp.zeros((LANES,), jnp.int32)
    idx_i[pl.ds(B_PER_W, LANES)] = jnp.zeros((LANES,), jnp.int32)

    w0 = wb_v[pl.ds(0, LANES)]
    w1 = wb_v[pl.ds(LANES, LANES)]
    bias = wb_v[pl.ds(2 * LANES, LANES)]
    lanes = lax.iota(jnp.int32, LANES)

    def issue_group(g, slot):
        base = g * G
        uvec = idx_u[pl.ds(base, 16)]
        ivec = idx_i[pl.ds(base, 16)]
        for r in range(G):
            cu = pl.multiple_of((uvec[r] >> 7) << 7, TCOL)
            ci = pl.multiple_of((ivec[r] >> 7) << 7, TCOL)
            pltpu.async_copy(tab_u_hbm.at[:, pl.ds(cu, TCOL)],
                             ubuf.at[slot, r], sem_u)
            pltpu.async_copy(tab_i_hbm.at[:, pl.ds(ci, TCOL)],
                             ibuf.at[slot, r], sem_i)

    def wait_group(slot):
        for r in range(G):
            pltpu.make_async_copy(tab_u_hbm.at[:, pl.ds(0, TCOL)],
                                  ubuf.at[slot, r], sem_u).wait()
            pltpu.make_async_copy(tab_i_hbm.at[:, pl.ds(0, TCOL)],
                                  ibuf.at[slot, r], sem_i).wait()

    issue_group(0, 0)

    def group_body(g, acc):
        slot = g % 2
        base = g * G

        @pl.when(g + 1 < NGROUP)
        def _issue():
            issue_group(g + 1, (g + 1) % 2)

        wait_group(slot)

        svec = jnp.full((LANES,), slot, jnp.int32)
        lu_all = idx_u[pl.ds(base, 16)] & (TCOL - 1)
        li_all = idx_i[pl.ds(base, 16)] & (TCOL - 1)
        for r in range(G):
            rvec = jnp.full((LANES,), r, jnp.int32)
            lu = jnp.full((LANES,), lu_all[r], jnp.int32)
            li = jnp.full((LANES,), li_all[r], jnp.int32)
            u0 = plsc.load_gather(ubuf, [svec, rvec, lanes, lu])
            u1 = plsc.load_gather(ubuf, [svec, rvec, lanes + LANES, lu])
            v0 = plsc.load_gather(ibuf, [svec, rvec, lanes, li])
            v1 = plsc.load_gather(ibuf, [svec, rvec, lanes + LANES, li])
            p = u0 * v0 * w0 + u1 * v1 * w1
            s = jnp.sum(p, axis=0)
            acc = jnp.where(lanes == (g * G + r) % LANES, s, acc)

        @pl.when((g + 1) % (LANES // G) == 0)
        def _flush():
            t = acc + bias
            out_v[pl.ds((g // (LANES // G)) * LANES, LANES)] = (
                1.0 / (1.0 + jnp.exp(-t)))
        return acc

    lax.fori_loop(0, NGROUP, group_body, jnp.zeros((LANES,), jnp.float32))

    pltpu.sync_copy(out_v, out_hbm.at[pl.ds(wid * B_PER_W, B_PER_W)])


@jax.jit
def _gmf(users, items, table_u_t, table_i_t, wb):
    mesh = plsc.VectorSubcoreMesh(
        core_axis_name="c", subcore_axis_name="s",
        num_cores=NC, num_subcores=NS)
    out = pl.kernel(
        _gmf_body,
        out_type=jax.ShapeDtypeStruct((BATCH,), jnp.float32),
        mesh=mesh,
        scratch_types=[
            pltpu.VMEM((B_PER_W + LANES,), jnp.int32),
            pltpu.VMEM((B_PER_W + LANES,), jnp.int32),
            pltpu.VMEM((2, G, NUM_FACTOR, TCOL), jnp.float32),
            pltpu.VMEM((2, G, NUM_FACTOR, TCOL), jnp.float32),
            pltpu.VMEM((B_PER_W,), jnp.float32),
            pltpu.VMEM((3 * LANES,), jnp.float32),
            pltpu.SemaphoreType.DMA,
            pltpu.SemaphoreType.DMA,
        ],
        compiler_params=pltpu.CompilerParams(needs_layout_passes=False),
    )(users, items, table_u_t, table_i_t, wb)
    return out


def kernel(users, items, user_table, item_table, W, b):
    wb = jnp.concatenate([W.reshape(-1), jnp.broadcast_to(b, (LANES,))])
    out = _gmf(users.astype(jnp.int32).reshape(NW, B_PER_W),
               items.astype(jnp.int32).reshape(NW, B_PER_W),
               user_table.T, item_table.T, wb)
    return out.reshape(BATCH, 1)


# speedup vs baseline: 1.0919x; 1.0919x over previous
import jax
import jax.numpy as jnp
from jax import lax
from jax.experimental import pallas as pl
from jax.experimental.pallas import tpu as pltpu
from jax.experimental.pallas import tpu_sc as plsc

NUM_FACTOR = 32
BATCH = 16384
NC = 2
NS = 16
LANES = 16
NW = NC * NS
B_PER_W = BATCH // NW
G = 4
NGROUP = B_PER_W // G
TCOL = 128


def _gmf_body(users_hbm, items_hbm, tab_u_hbm, tab_i_hbm, wb_hbm,
              out_hbm, idx_u, idx_i, ubuf, ibuf, out_v, wb_v,
              sem_u, sem_i):
    wid = lax.axis_index("s") * NC + lax.axis_index("c")

    pltpu.sync_copy(users_hbm.at[wid], idx_u.at[pl.ds(0, B_PER_W)])
    pltpu.sync_copy(items_hbm.at[wid], idx_i.at[pl.ds(0, B_PER_W)])
    pltpu.sync_copy(wb_hbm, wb_v)
    idx_u[pl.ds(B_PER_W, LANES)] = jnp.zeros((LANES,), jnp.int32)
    idx_i[pl.ds(B_PER_W, LANES)] = jnp.zeros((LANES,), jnp.int32)

    w0 = wb_v[pl.ds(0, LANES)]
    w1 = wb_v[pl.ds(LANES, LANES)]
    bias = wb_v[pl.ds(2 * LANES, LANES)]
    lanes = lax.iota(jnp.int32, LANES)

    def issue_group(g, slot):
        base = g * G
        uvec = idx_u[pl.ds(base, 16)]
        ivec = idx_i[pl.ds(base, 16)]
        for r in range(G):
            cu = pl.multiple_of((uvec[r] >> 7) << 7, TCOL)
            ci = pl.multiple_of((ivec[r] >> 7) << 7, TCOL)
            pltpu.async_copy(tab_u_hbm.at[:, pl.ds(cu, TCOL)],
                             ubuf.at[slot, r], sem_u)
            pltpu.async_copy(tab_i_hbm.at[:, pl.ds(ci, TCOL)],
                             ibuf.at[slot, r], sem_i)

    def wait_group(slot):
        for r in range(G):
            pltpu.make_async_copy(tab_u_hbm.at[:, pl.ds(0, TCOL)],
                                  ubuf.at[slot, r], sem_u).wait()
            pltpu.make_async_copy(tab_i_hbm.at[:, pl.ds(0, TCOL)],
                                  ibuf.at[slot, r], sem_i).wait()

    issue_group(0, 0)
    issue_group(1, 1)

    def group_body(g, acc):
        slot = g % 3
        base = g * G

        @pl.when(g + 2 < NGROUP)
        def _issue():
            issue_group(g + 2, (g + 2) % 3)

        wait_group(slot)

        svec = jnp.full((LANES,), slot, jnp.int32)
        lu_all = idx_u[pl.ds(base, 16)] & (TCOL - 1)
        li_all = idx_i[pl.ds(base, 16)] & (TCOL - 1)
        for r in range(G):
            rvec = jnp.full((LANES,), r, jnp.int32)
            lu = jnp.full((LANES,), lu_all[r], jnp.int32)
            li = jnp.full((LANES,), li_all[r], jnp.int32)
            u0 = plsc.load_gather(ubuf, [svec, rvec, lanes, lu])
            u1 = plsc.load_gather(ubuf, [svec, rvec, lanes + LANES, lu])
            v0 = plsc.load_gather(ibuf, [svec, rvec, lanes, li])
            v1 = plsc.load_gather(ibuf, [svec, rvec, lanes + LANES, li])
            p = u0 * v0 * w0 + u1 * v1 * w1
            s = jnp.sum(p, axis=0)
            acc = jnp.where(lanes == (g * G + r) % LANES, s, acc)

        @pl.when((g + 1) % (LANES // G) == 0)
        def _flush():
            t = acc + bias
            out_v[pl.ds((g // (LANES // G)) * LANES, LANES)] = (
                1.0 / (1.0 + jnp.exp(-t)))
        return acc

    lax.fori_loop(0, NGROUP, group_body, jnp.zeros((LANES,), jnp.float32))

    pltpu.sync_copy(out_v, out_hbm.at[pl.ds(wid * B_PER_W, B_PER_W)])


@jax.jit
def _gmf(users, items, table_u_t, table_i_t, wb):
    mesh = plsc.VectorSubcoreMesh(
        core_axis_name="c", subcore_axis_name="s",
        num_cores=NC, num_subcores=NS)
    out = pl.kernel(
        _gmf_body,
        out_type=jax.ShapeDtypeStruct((BATCH,), jnp.float32),
        mesh=mesh,
        scratch_types=[
            pltpu.VMEM((B_PER_W + LANES,), jnp.int32),
            pltpu.VMEM((B_PER_W + LANES,), jnp.int32),
            pltpu.VMEM((3, G, NUM_FACTOR, TCOL), jnp.float32),
            pltpu.VMEM((3, G, NUM_FACTOR, TCOL), jnp.float32),
            pltpu.VMEM((B_PER_W,), jnp.float32),
            pltpu.VMEM((3 * LANES,), jnp.float32),
            pltpu.SemaphoreType.DMA,
            pltpu.SemaphoreType.DMA,
        ],
        compiler_params=pltpu.CompilerParams(needs_layout_passes=False),
    )(users, items, table_u_t, table_i_t, wb)
    return out


def kernel(users, items, user_table, item_table, W, b):
    wb = jnp.concatenate([W.reshape(-1), jnp.broadcast_to(b, (LANES,))])
    out = _gmf(users.astype(jnp.int32).reshape(NW, B_PER_W),
               items.astype(jnp.int32).reshape(NW, B_PER_W),
               user_table.T, item_table.T, wb)
    return out.reshape(BATCH, 1)


# speedup vs baseline: 1.0929x; 1.0009x over previous
import jax
import jax.numpy as jnp
from jax import lax
from jax.experimental import pallas as pl
from jax.experimental.pallas import tpu as pltpu
from jax.experimental.pallas import tpu_sc as plsc

NUM_FACTOR = 32
BATCH = 16384
NC = 2
NS = 16
LANES = 16
NW = NC * NS
B_PER_W = BATCH // NW
G = 2
NBUF = 6
NGROUP = B_PER_W // G
TCOL = 128


def _gmf_body(users_hbm, items_hbm, tab_u_hbm, tab_i_hbm, wb_hbm,
              out_hbm, idx_u, idx_i, ubuf, ibuf, out_v, wb_v,
              sem_u, sem_i):
    wid = lax.axis_index("s") * NC + lax.axis_index("c")

    pltpu.sync_copy(users_hbm.at[wid], idx_u.at[pl.ds(0, B_PER_W)])
    pltpu.sync_copy(items_hbm.at[wid], idx_i.at[pl.ds(0, B_PER_W)])
    pltpu.sync_copy(wb_hbm, wb_v)
    idx_u[pl.ds(B_PER_W, LANES)] = jnp.zeros((LANES,), jnp.int32)
    idx_i[pl.ds(B_PER_W, LANES)] = jnp.zeros((LANES,), jnp.int32)

    w0 = wb_v[pl.ds(0, LANES)]
    w1 = wb_v[pl.ds(LANES, LANES)]
    bias = wb_v[pl.ds(2 * LANES, LANES)]
    lanes = lax.iota(jnp.int32, LANES)

    def issue_group(g, slot):
        base = g * G
        uvec = idx_u[pl.ds(base, 16)]
        ivec = idx_i[pl.ds(base, 16)]
        for r in range(G):
            cu = pl.multiple_of((uvec[r] >> 7) << 7, TCOL)
            ci = pl.multiple_of((ivec[r] >> 7) << 7, TCOL)
            pltpu.async_copy(tab_u_hbm.at[:, pl.ds(cu, TCOL)],
                             ubuf.at[slot, r], sem_u)
            pltpu.async_copy(tab_i_hbm.at[:, pl.ds(ci, TCOL)],
                             ibuf.at[slot, r], sem_i)

    def wait_group(slot):
        for r in range(G):
            pltpu.make_async_copy(tab_u_hbm.at[:, pl.ds(0, TCOL)],
                                  ubuf.at[slot, r], sem_u).wait()
            pltpu.make_async_copy(tab_i_hbm.at[:, pl.ds(0, TCOL)],
                                  ibuf.at[slot, r], sem_i).wait()

    for p in range(NBUF - 1):
        issue_group(p, p)

    def group_body(g, acc):
        slot = g % NBUF
        base = g * G

        @pl.when(g + NBUF - 1 < NGROUP)
        def _issue():
            issue_group(g + NBUF - 1, (g + NBUF - 1) % NBUF)

        wait_group(slot)

        svec = jnp.full((LANES,), slot, jnp.int32)
        lu_all = idx_u[pl.ds(base, 16)] & (TCOL - 1)
        li_all = idx_i[pl.ds(base, 16)] & (TCOL - 1)
        for r in range(G):
            rvec = jnp.full((LANES,), r, jnp.int32)
            lu = jnp.full((LANES,), lu_all[r], jnp.int32)
            li = jnp.full((LANES,), li_all[r], jnp.int32)
            u0 = plsc.load_gather(ubuf, [svec, rvec, lanes, lu])
            u1 = plsc.load_gather(ubuf, [svec, rvec, lanes + LANES, lu])
            v0 = plsc.load_gather(ibuf, [svec, rvec, lanes, li])
            v1 = plsc.load_gather(ibuf, [svec, rvec, lanes + LANES, li])
            p = u0 * v0 * w0 + u1 * v1 * w1
            s = jnp.sum(p, axis=0)
            acc = jnp.where(lanes == (g * G + r) % LANES, s, acc)

        @pl.when((g + 1) % (LANES // G) == 0)
        def _flush():
            t = acc + bias
            out_v[pl.ds((g // (LANES // G)) * LANES, LANES)] = (
                1.0 / (1.0 + jnp.exp(-t)))
        return acc

    lax.fori_loop(0, NGROUP, group_body, jnp.zeros((LANES,), jnp.float32))

    pltpu.sync_copy(out_v, out_hbm.at[pl.ds(wid * B_PER_W, B_PER_W)])


@jax.jit
def _gmf(users, items, table_u_t, table_i_t, wb):
    mesh = plsc.VectorSubcoreMesh(
        core_axis_name="c", subcore_axis_name="s",
        num_cores=NC, num_subcores=NS)
    out = pl.kernel(
        _gmf_body,
        out_type=jax.ShapeDtypeStruct((BATCH,), jnp.float32),
        mesh=mesh,
        scratch_types=[
            pltpu.VMEM((B_PER_W + LANES,), jnp.int32),
            pltpu.VMEM((B_PER_W + LANES,), jnp.int32),
            pltpu.VMEM((NBUF, G, NUM_FACTOR, TCOL), jnp.float32),
            pltpu.VMEM((NBUF, G, NUM_FACTOR, TCOL), jnp.float32),
            pltpu.VMEM((B_PER_W,), jnp.float32),
            pltpu.VMEM((3 * LANES,), jnp.float32),
            pltpu.SemaphoreType.DMA,
            pltpu.SemaphoreType.DMA,
        ],
        compiler_params=pltpu.CompilerParams(needs_layout_passes=False),
    )(users, items, table_u_t, table_i_t, wb)
    return out


def kernel(users, items, user_table, item_table, W, b):
    wb = jnp.concatenate([W.reshape(-1), jnp.broadcast_to(b, (LANES,))])
    out = _gmf(users.astype(jnp.int32).reshape(NW, B_PER_W),
               items.astype(jnp.int32).reshape(NW, B_PER_W),
               user_table.T, item_table.T, wb)
    return out.reshape(BATCH, 1)


# speedup vs baseline: 1.3431x; 1.2290x over previous
import jax
import jax.numpy as jnp
from jax import lax
from jax.experimental import pallas as pl
from jax.experimental.pallas import tpu as pltpu
from jax.experimental.pallas import tpu_sc as plsc

NUM_FACTOR = 32
NUM_ITEMS = 100000
BATCH = 16384
NC = 2
NS = 16
LANES = 16
NW = NC * NS
B_PER_W = BATCH // NW
G = 2
NBUF = 6
NGROUP = B_PER_W // G
TCOL = 128
ITEM_COLS = (NUM_ITEMS + TCOL - 1) // TCOL
NCOLS = 25
MAXM = 768
VW_ROWS = BATCH + 256


def _item_body(items_hbm, tab_i_hbm, w_hbm, vw_hbm,
               itv, colbuf, tlist, plist, plist2, vwbuf, wv, sem):
    wid = lax.axis_index("s") * NC + lax.axis_index("c")
    cstart = wid * ITEM_COLS // NW
    cend = (wid + 1) * ITEM_COLS // NW
    lo = cstart * TCOL
    hi = jnp.minimum(cend * TCOL, NUM_ITEMS)
    cbase = jnp.minimum(cstart, ITEM_COLS - NCOLS)

    pltpu.sync_copy(w_hbm, wv)
    pltpu.async_copy(
        tab_i_hbm.at[:, pl.ds(pl.multiple_of(cbase * TCOL, TCOL),
                              NCOLS * TCOL)],
        colbuf, sem).wait()

    w0 = wv[pl.ds(0, LANES)]
    w1 = wv[pl.ds(LANES, LANES)]
    lanes = lax.iota(jnp.int32, LANES)

    def fill_body(k, _):
        tlist[pl.ds(k * LANES, LANES)] = jnp.full((LANES,), lo, jnp.int32)
        plist[pl.ds(k * LANES, LANES)] = (
            BATCH + lanes + (k % 8) * LANES)
        return _
    lax.fori_loop(0, MAXM // LANES, fill_body, 0)

    def outer_scan(c, off):
        pltpu.sync_copy(items_hbm.at[pl.ds(c * 1024, 1024)], itv)
        def scan_body(i, off):
            tv = itv[pl.ds(i * LANES, LANES)]
            m = (tv >= lo) & (tv < hi)
            pv = lanes + (c * 1024 + i * LANES)
            plsc.store_compressed(tlist.at[pl.ds(off, LANES)], tv, mask=m)
            plsc.store_compressed(plist.at[pl.ds(off, LANES)], pv, mask=m)
            cnt = plsc.all_reduce_population_count(m)
            return off + cnt[0]
        return lax.fori_loop(0, 1024 // LANES, scan_body, off)
    lax.fori_loop(0, BATCH // 1024, outer_scan, 0)

    for k in range(MAXM // TCOL):
        for rr in range(TCOL // LANES):
            plist2[k, pl.ds(rr * LANES, LANES)] = (
                plist[pl.ds(k * TCOL + rr * LANES, LANES)])

    for k in range(MAXM // TCOL):
        def chunk_body(rr, _):
            tvec = tlist[pl.ds(k * TCOL + rr * LANES, LANES)]
            tloc = tvec - cbase * TCOL
            for r in range(LANES):
                tl = jnp.full((LANES,), tloc[r], jnp.int32)
                g0 = plsc.load_gather(colbuf, [lanes, tl])
                g1 = plsc.load_gather(colbuf, [lanes + LANES, tl])
                row = rr * LANES + r
                vwbuf[row, pl.ds(0, LANES)] = g0 * w0
                vwbuf[row, pl.ds(LANES, LANES)] = g1 * w1
            return _
        lax.fori_loop(0, TCOL // LANES, chunk_body, 0)
        pltpu.async_copy(vwbuf, vw_hbm.at[plist2.at[k]], sem).wait()


def _user_body(users_hbm, tab_u_hbm, vw_hbm, b_hbm,
               out_hbm, idx_u, vwv, ubuf, out_v, b_v, sem_u, sem_v):
    wid = lax.axis_index("s") * NC + lax.axis_index("c")

    pltpu.sync_copy(users_hbm.at[wid], idx_u.at[pl.ds(0, B_PER_W)])
    pltpu.sync_copy(b_hbm, b_v)
    idx_u[pl.ds(B_PER_W, LANES)] = jnp.zeros((LANES,), jnp.int32)
    pltpu.async_copy(vw_hbm.at[pl.ds(wid * B_PER_W, B_PER_W)], vwv,
                     sem_v).wait()

    bias = b_v[pl.ds(0, LANES)]
    lanes = lax.iota(jnp.int32, LANES)

    def issue_group(g, slot):
        base = g * G
        uvec = idx_u[pl.ds(base, 16)]
        for r in range(G):
            cu = pl.multiple_of((uvec[r] >> 7) << 7, TCOL)
            pltpu.async_copy(tab_u_hbm.at[:, pl.ds(cu, TCOL)],
                             ubuf.at[slot, r], sem_u)

    def wait_group(slot):
        for r in range(G):
            pltpu.make_async_copy(tab_u_hbm.at[:, pl.ds(0, TCOL)],
                                  ubuf.at[slot, r], sem_u).wait()

    for p in range(NBUF - 1):
        issue_group(p, p)

    def group_body(g, acc):
        slot = g % NBUF
        base = g * G

        @pl.when(g + NBUF - 1 < NGROUP)
        def _issue():
            issue_group(g + NBUF - 1, (g + NBUF - 1) % NBUF)

        wait_group(slot)

        svec = jnp.full((LANES,), slot, jnp.int32)
        lu_all = idx_u[pl.ds(base, 16)] & (TCOL - 1)
        for r in range(G):
            rvec = jnp.full((LANES,), r, jnp.int32)
            lu = jnp.full((LANES,), lu_all[r], jnp.int32)
            u0 = plsc.load_gather(ubuf, [svec, rvec, lanes, lu])
            u1 = plsc.load_gather(ubuf, [svec, rvec, lanes + LANES, lu])
            vw0 = vwv[base + r, pl.ds(0, LANES)]
            vw1 = vwv[base + r, pl.ds(LANES, LANES)]
            p = u0 * vw0 + u1 * vw1
            s = jnp.sum(p, axis=0)
            acc = jnp.where(lanes == (g * G + r) % LANES, s, acc)

        @pl.when((g + 1) % (LANES // G) == 0)
        def _flush():
            t = acc + bias
            out_v[pl.ds((g // (LANES // G)) * LANES, LANES)] = (
                1.0 / (1.0 + jnp.exp(-t)))
        return acc

    lax.fori_loop(0, NGROUP, group_body, jnp.zeros((LANES,), jnp.float32))

    pltpu.sync_copy(out_v, out_hbm.at[pl.ds(wid * B_PER_W, B_PER_W)])


@jax.jit
def _gmf(users, items_flat, table_u_t, table_i_t, w_flat, b_bcast):
    mesh = plsc.VectorSubcoreMesh(
        core_axis_name="c", subcore_axis_name="s",
        num_cores=NC, num_subcores=NS)
    vw = pl.kernel(
        _item_body,
        out_type=jax.ShapeDtypeStruct((VW_ROWS, TCOL), jnp.float32),
        mesh=mesh,
        scratch_types=[
            pltpu.VMEM((1024,), jnp.int32),
            pltpu.VMEM((NUM_FACTOR, NCOLS * TCOL), jnp.float32),
            pltpu.VMEM((MAXM + LANES,), jnp.int32),
            pltpu.VMEM((MAXM + LANES,), jnp.int32),
            pltpu.VMEM((MAXM // TCOL, TCOL), jnp.int32),
            pltpu.VMEM((TCOL, TCOL), jnp.float32),
            pltpu.VMEM((2 * LANES,), jnp.float32),
            pltpu.SemaphoreType.DMA,
        ],
        compiler_params=pltpu.CompilerParams(needs_layout_passes=False),
    )(items_flat, table_i_t, w_flat)
    out = pl.kernel(
        _user_body,
        out_type=jax.ShapeDtypeStruct((BATCH,), jnp.float32),
        mesh=mesh,
        scratch_types=[
            pltpu.VMEM((B_PER_W + LANES,), jnp.int32),
            pltpu.VMEM((B_PER_W, TCOL), jnp.float32),
            pltpu.VMEM((NBUF, G, NUM_FACTOR, TCOL), jnp.float32),
            pltpu.VMEM((B_PER_W,), jnp.float32),
            pltpu.VMEM((LANES,), jnp.float32),
            pltpu.SemaphoreType.DMA,
            pltpu.SemaphoreType.DMA,
        ],
        compiler_params=pltpu.CompilerParams(needs_layout_passes=False),
    )(users.reshape(NW, B_PER_W), table_u_t, vw, b_bcast)
    return out


def kernel(users, items, user_table, item_table, W, b):
    out = _gmf(users.astype(jnp.int32), items.astype(jnp.int32),
               user_table.T, item_table.T,
               W.reshape(-1), jnp.broadcast_to(b, (LANES,)))
    return out.reshape(BATCH, 1)


# speedup vs baseline: 1.3874x; 1.0330x over previous
import jax
import jax.numpy as jnp
from jax import lax
from jax.experimental import pallas as pl
from jax.experimental.pallas import tpu as pltpu
from jax.experimental.pallas import tpu_sc as plsc

NUM_FACTOR = 32
NUM_ITEMS = 100000
BATCH = 16384
NC = 2
NS = 16
LANES = 16
NW = NC * NS
B_PER_W = BATCH // NW
G = 2
NBUF = 6
NGROUP = B_PER_W // G
TCOL = 128
ITEM_COLS = (NUM_ITEMS + TCOL - 1) // TCOL
NCOLS = 25
MAXM = 768
VW_ROWS = BATCH + 256


def _item_body(items_hbm, tab_i_hbm, w_hbm, vw_hbm,
               itv, colbuf, tlist, plist, plist2, vwbuf, wv, sem):
    wid = lax.axis_index("s") * NC + lax.axis_index("c")
    cstart = wid * ITEM_COLS // NW
    cend = (wid + 1) * ITEM_COLS // NW
    lo = cstart * TCOL
    hi = jnp.minimum(cend * TCOL, NUM_ITEMS)
    cbase = jnp.minimum(cstart, ITEM_COLS - NCOLS)

    pltpu.sync_copy(w_hbm, wv)
    col_copy = pltpu.async_copy(
        tab_i_hbm.at[:, pl.ds(pl.multiple_of(cbase * TCOL, TCOL),
                              NCOLS * TCOL)],
        colbuf, sem)

    w0 = wv[pl.ds(0, LANES)]
    w1 = wv[pl.ds(LANES, LANES)]
    lanes = lax.iota(jnp.int32, LANES)

    def fill_body(k, _):
        tlist[pl.ds(k * LANES, LANES)] = jnp.full((LANES,), lo, jnp.int32)
        plist[pl.ds(k * LANES, LANES)] = (
            BATCH + lanes + (k % 8) * LANES)
        return _
    lax.fori_loop(0, MAXM // LANES, fill_body, 0)

    def outer_scan(c, off):
        pltpu.sync_copy(items_hbm.at[pl.ds(c * 1024, 1024)], itv)
        def scan_body(i, off):
            tv = itv[pl.ds(i * LANES, LANES)]
            m = (tv >= lo) & (tv < hi)
            pv = lanes + (c * 1024 + i * LANES)
            plsc.store_compressed(tlist.at[pl.ds(off, LANES)], tv, mask=m)
            plsc.store_compressed(plist.at[pl.ds(off, LANES)], pv, mask=m)
            cnt = plsc.all_reduce_population_count(m)
            return off + cnt[0]
        return lax.fori_loop(0, 1024 // LANES, scan_body, off)
    nmatch = lax.fori_loop(0, BATCH // 1024, outer_scan, 0)
    col_copy.wait()

    for k in range(MAXM // TCOL):
        for rr in range(TCOL // LANES):
            plist2[k, pl.ds(rr * LANES, LANES)] = (
                plist[pl.ds(k * TCOL + rr * LANES, LANES)])

    for k in range(MAXM // TCOL):
        @pl.when(k * TCOL < nmatch)
        def _do_chunk(k=k):
            def chunk_body(rr, _):
                tvec = tlist[pl.ds(k * TCOL + rr * LANES, LANES)]
                tloc = tvec - cbase * TCOL
                for r in range(LANES):
                    tl = jnp.full((LANES,), tloc[r], jnp.int32)
                    g0 = plsc.load_gather(colbuf, [lanes, tl])
                    g1 = plsc.load_gather(colbuf, [lanes + LANES, tl])
                    row = rr * LANES + r
                    vwbuf[row, pl.ds(0, LANES)] = g0 * w0
                    vwbuf[row, pl.ds(LANES, LANES)] = g1 * w1
                return _
            lax.fori_loop(0, TCOL // LANES, chunk_body, 0)
            pltpu.async_copy(vwbuf, vw_hbm.at[plist2.at[k]], sem).wait()


def _user_body(users_hbm, tab_u_hbm, vw_hbm, b_hbm,
               out_hbm, idx_u, vwv, ubuf, out_v, b_v, sem_u, sem_v):
    wid = lax.axis_index("s") * NC + lax.axis_index("c")

    pltpu.sync_copy(users_hbm.at[wid], idx_u.at[pl.ds(0, B_PER_W)])
    pltpu.sync_copy(b_hbm, b_v)
    idx_u[pl.ds(B_PER_W, LANES)] = jnp.zeros((LANES,), jnp.int32)
    pltpu.async_copy(vw_hbm.at[pl.ds(wid * B_PER_W, B_PER_W)], vwv,
                     sem_v).wait()

    bias = b_v[pl.ds(0, LANES)]
    lanes = lax.iota(jnp.int32, LANES)

    def issue_group(g, slot):
        base = g * G
        uvec = idx_u[pl.ds(base, 16)]
        for r in range(G):
            cu = pl.multiple_of((uvec[r] >> 7) << 7, TCOL)
            pltpu.async_copy(tab_u_hbm.at[:, pl.ds(cu, TCOL)],
                             ubuf.at[slot, r], sem_u)

    def wait_group(slot):
        for r in range(G):
            pltpu.make_async_copy(tab_u_hbm.at[:, pl.ds(0, TCOL)],
                                  ubuf.at[slot, r], sem_u).wait()

    for p in range(NBUF - 1):
        issue_group(p, p)

    def group_body(g, acc):
        slot = g % NBUF
        base = g * G

        @pl.when(g + NBUF - 1 < NGROUP)
        def _issue():
            issue_group(g + NBUF - 1, (g + NBUF - 1) % NBUF)

        wait_group(slot)

        svec = jnp.full((LANES,), slot, jnp.int32)
        lu_all = idx_u[pl.ds(base, 16)] & (TCOL - 1)
        for r in range(G):
            rvec = jnp.full((LANES,), r, jnp.int32)
            lu = jnp.full((LANES,), lu_all[r], jnp.int32)
            u0 = plsc.load_gather(ubuf, [svec, rvec, lanes, lu])
            u1 = plsc.load_gather(ubuf, [svec, rvec, lanes + LANES, lu])
            vw0 = vwv[base + r, pl.ds(0, LANES)]
            vw1 = vwv[base + r, pl.ds(LANES, LANES)]
            p = u0 * vw0 + u1 * vw1
            s = jnp.sum(p, axis=0)
            acc = jnp.where(lanes == (g * G + r) % LANES, s, acc)

        @pl.when((g + 1) % (LANES // G) == 0)
        def _flush():
            t = acc + bias
            out_v[pl.ds((g // (LANES // G)) * LANES, LANES)] = (
                1.0 / (1.0 + jnp.exp(-t)))
        return acc

    lax.fori_loop(0, NGROUP, group_body, jnp.zeros((LANES,), jnp.float32))

    pltpu.sync_copy(out_v, out_hbm.at[pl.ds(wid * B_PER_W, B_PER_W)])


@jax.jit
def _gmf(users, items_flat, table_u_t, table_i_t, w_flat, b_bcast):
    mesh = plsc.VectorSubcoreMesh(
        core_axis_name="c", subcore_axis_name="s",
        num_cores=NC, num_subcores=NS)
    vw = pl.kernel(
        _item_body,
        out_type=jax.ShapeDtypeStruct((VW_ROWS, TCOL), jnp.float32),
        mesh=mesh,
        scratch_types=[
            pltpu.VMEM((1024,), jnp.int32),
            pltpu.VMEM((NUM_FACTOR, NCOLS * TCOL), jnp.float32),
            pltpu.VMEM((MAXM + LANES,), jnp.int32),
            pltpu.VMEM((MAXM + LANES,), jnp.int32),
            pltpu.VMEM((MAXM // TCOL, TCOL), jnp.int32),
            pltpu.VMEM((TCOL, TCOL), jnp.float32),
            pltpu.VMEM((2 * LANES,), jnp.float32),
            pltpu.SemaphoreType.DMA,
        ],
        compiler_params=pltpu.CompilerParams(needs_layout_passes=False),
    )(items_flat, table_i_t, w_flat)
    out = pl.kernel(
        _user_body,
        out_type=jax.ShapeDtypeStruct((BATCH,), jnp.float32),
        mesh=mesh,
        scratch_types=[
            pltpu.VMEM((B_PER_W + LANES,), jnp.int32),
            pltpu.VMEM((B_PER_W, TCOL), jnp.float32),
            pltpu.VMEM((NBUF, G, NUM_FACTOR, TCOL), jnp.float32),
            pltpu.VMEM((B_PER_W,), jnp.float32),
            pltpu.VMEM((LANES,), jnp.float32),
            pltpu.SemaphoreType.DMA,
            pltpu.SemaphoreType.DMA,
        ],
        compiler_params=pltpu.CompilerParams(needs_layout_passes=False),
    )(users.reshape(NW, B_PER_W), table_u_t, vw, b_bcast)
    return out


def kernel(users, items, user_table, item_table, W, b):
    out = _gmf(users.astype(jnp.int32), items.astype(jnp.int32),
               user_table.T, item_table.T,
               W.reshape(-1), jnp.broadcast_to(b, (LANES,)))
    return out.reshape(BATCH, 1)
